# baseline (speedup 1.0000x reference)
import functools

import jax
import jax.numpy as jnp
from jax import lax
from jax.experimental import pallas as pl
from jax.experimental.pallas import tpu as pltpu
from jax.experimental.pallas import tpu_sc as plsc

_NC, _NS = 2, 16
_NW = _NC * _NS

_BT = 256
_BTG = 512
_GC = 80
_CC = 16


def _gate_body(e_static, x_ref, wg_ref, e0_ref, e1_ref, w0_ref, w1_ref):
    logits = jnp.dot(x_ref[...], wg_ref[...], preferred_element_type=jnp.float32)
    col = lax.broadcasted_iota(jnp.int32, logits.shape, 1)
    neg = jnp.float32(-1e30)
    l0 = jnp.where(col < e_static, logits, neg)
    m1 = jnp.max(l0, axis=1, keepdims=True)
    i1 = jnp.min(jnp.where(l0 == m1, col, logits.shape[1]), axis=1, keepdims=True)
    lm = jnp.where(col == i1, neg, l0)
    m2 = jnp.max(lm, axis=1, keepdims=True)
    i2 = jnp.min(jnp.where(lm == m2, col, logits.shape[1]), axis=1, keepdims=True)
    t = jnp.exp(m2 - m1)
    w0 = 1.0 / (1.0 + t)
    e0_ref[0, 0, :] = i1[:, 0]
    e1_ref[0, 0, :] = i2[:, 0]
    w0_ref[0, 0, :] = w0[:, 0]
    w1_ref[0, 0, :] = 1.0 - w0[:, 0]


def _ffn_body(be_ref, xs_ref, w1_ref, b1_ref, w2_ref, b2_ref, ws_ref, ys_ref):
    xb = xs_ref[...]
    h = jnp.dot(xb, w1_ref[0], preferred_element_type=jnp.float32) + b1_ref[0]
    g = jax.nn.gelu(h)
    y = jnp.dot(g.astype(jnp.bfloat16), w2_ref[0],
                preferred_element_type=jnp.float32) + b2_ref[0]
    ys_ref[...] = y * ws_ref[0, 0][:, None]


def _gating(xf, Wg):
    t, d = xf.shape
    e = Wg.shape[1]
    lanes = 128
    wg_pad = jnp.pad(Wg, ((0, 0), (0, lanes - e)))
    nbg = t // _BTG
    e0, e1, w0, w1 = pl.pallas_call(
        functools.partial(_gate_body, e),
        grid=(nbg,),
        in_specs=[
            pl.BlockSpec((_BTG, d), lambda i: (i, 0)),
            pl.BlockSpec((d, lanes), lambda i: (0, 0)),
        ],
        out_specs=[
            pl.BlockSpec((1, 1, _BTG), lambda i: (i, 0, 0)),
            pl.BlockSpec((1, 1, _BTG), lambda i: (i, 0, 0)),
            pl.BlockSpec((1, 1, _BTG), lambda i: (i, 0, 0)),
            pl.BlockSpec((1, 1, _BTG), lambda i: (i, 0, 0)),
        ],
        out_shape=[
            jax.ShapeDtypeStruct((nbg, 1, _BTG), jnp.int32),
            jax.ShapeDtypeStruct((nbg, 1, _BTG), jnp.int32),
            jax.ShapeDtypeStruct((nbg, 1, _BTG), jnp.float32),
            jax.ShapeDtypeStruct((nbg, 1, _BTG), jnp.float32),
        ],
    )(xf, wg_pad)
    return e0.reshape(t), e1.reshape(t), w0.reshape(t), w1.reshape(t)


def _route(e0, e1, w0, w1, e):
    t = e0.shape[0]
    p = t * 2
    nb = p // _BT + e
    npos = nb * _BT
    flat_e = jnp.stack([e0, e1], axis=1).reshape(p)
    flat_w = jnp.stack([w0, w1], axis=1).reshape(p)
    oh = (flat_e[:, None] == jnp.arange(e, dtype=jnp.int32)[None, :]).astype(jnp.int32)
    csum = jnp.cumsum(oh, axis=0)
    counts = csum[-1]
    rank = jnp.take_along_axis(csum, flat_e[:, None], axis=1)[:, 0] - 1
    nblk = (counts + _BT - 1) // _BT
    blk_start = jnp.concatenate(
        [jnp.zeros((1,), jnp.int32), jnp.cumsum(nblk)[:-1].astype(jnp.int32)])
    pos = blk_start[flat_e] * _BT + rank
    tok = jnp.arange(p, dtype=jnp.int32) // 2
    tok_pos = jnp.zeros((npos,), jnp.int32).at[pos].set(tok)
    ws_pos = jnp.zeros((npos,), jnp.float32).at[pos].set(flat_w)
    block_expert = jnp.minimum(
        (jnp.arange(nb, dtype=jnp.int32)[:, None] >= blk_start[None, :])
        .astype(jnp.int32).sum(axis=1) - 1,
        e - 1).astype(jnp.int32)
    return tok_pos, ws_pos, block_expert, pos


def _ffn(block_expert, xs, W1, b1, W2, b2, ws_pos):
    npos, d = xs.shape
    e, _, f = W1.shape
    nb = npos // _BT
    ws3 = ws_pos.reshape(nb, 1, _BT)
    grid_spec = pltpu.PrefetchScalarGridSpec(
        num_scalar_prefetch=1,
        grid=(nb,),
        in_specs=[
            pl.BlockSpec((_BT, d), lambda i, be: (i, 0)),
            pl.BlockSpec((1, d, f), lambda i, be: (be[i], 0, 0)),
            pl.BlockSpec((1, 1, f), lambda i, be: (be[i], 0, 0)),
            pl.BlockSpec((1, f, d), lambda i, be: (be[i], 0, 0)),
            pl.BlockSpec((1, 1, d), lambda i, be: (be[i], 0, 0)),
            pl.BlockSpec((1, 1, _BT), lambda i, be: (i, 0, 0)),
        ],
        out_specs=pl.BlockSpec((_BT, d), lambda i, be: (i, 0)),
    )
    return pl.pallas_call(
        _ffn_body,
        grid_spec=grid_spec,
        out_shape=jax.ShapeDtypeStruct((npos, d), jnp.float32),
    )(block_expert, xs, W1.astype(jnp.bfloat16), b1.reshape(e, 1, f),
      W2.astype(jnp.bfloat16), b2.reshape(e, 1, d), ws3)


def kernel(x, Wg, W1, b1, W2, b2):
    b, s, d = x.shape
    t = b * s
    e = Wg.shape[1]
    p = t * 2
    nb = p // _BT + e
    npos = nb * _BT

    xf = jnp.reshape(x, (t, d))

    e0, e1, w0, w1 = _gating(xf, Wg)

    tok_pos, ws_pos, block_expert, pos = _route(e0, e1, w0, w1, e)

    d2 = d // 2
    xb16 = xf.astype(jnp.bfloat16)
    x32 = lax.bitcast_convert_type(xb16.reshape(t, d2, 2), jnp.int32)
    mesh = plsc.VectorSubcoreMesh(
        core_axis_name="c", subcore_axis_name="s",
        num_cores=_NC, num_subcores=_NS)
    per_w = npos // _NW
    nch = per_w // _GC

    @functools.partial(
        pl.kernel,
        mesh=mesh,
        out_type=jax.ShapeDtypeStruct((npos, d2), jnp.int32),
        scratch_types=[
            pltpu.VMEM((per_w,), jnp.int32),
            pltpu.VMEM((_GC, d2), jnp.int32),
            pltpu.VMEM((_GC, d2), jnp.int32),
            pltpu.SemaphoreType.DMA,
            pltpu.SemaphoreType.DMA,
            pltpu.SemaphoreType.DMA,
            pltpu.SemaphoreType.DMA,
        ],
    )
    def dispatch(x_hbm, idx_hbm, xs_hbm, idx_v, buf0, buf1, g0, g1, w0s, w1s):
        wid = lax.axis_index("s") * _NC + lax.axis_index("c")
        base = wid * per_w
        pltpu.sync_copy(idx_hbm.at[pl.ds(base, per_w)], idx_v)
        bufs = (buf0, buf1)
        gsem = (g0, g1)
        wsem = (w0s, w1s)
        gcp = [None, None]
        wcp = [None, None]
        for c in range(nch):
            p = c & 1
            if wcp[p] is not None:
                wcp[p].wait()
            gcp[p] = pltpu.async_copy(
                x_hbm.at[idx_v.at[pl.ds(c * _GC, _GC)]], bufs[p], gsem[p])
            if c >= 1:
                q = 1 - p
                gcp[q].wait()
                wcp[q] = pltpu.async_copy(
                    bufs[q], xs_hbm.at[pl.ds(base + (c - 1) * _GC, _GC)], wsem[q])
        pl_ = (nch - 1) & 1
        gcp[pl_].wait()
        wcp[pl_] = pltpu.async_copy(
            bufs[pl_], xs_hbm.at[pl.ds(base + (nch - 1) * _GC, _GC)], wsem[pl_])
        for p in (0, 1):
            if wcp[p] is not None:
                wcp[p].wait()

    xs32 = dispatch(x32, tok_pos)
    xs = lax.bitcast_convert_type(xs32, jnp.bfloat16).reshape(npos, d)

    ys = _ffn(block_expert, xs, W1, b1, W2, b2, ws_pos)

    tpw = t // _NW
    ncc = tpw // _CC
    nvec = d // 16

    @functools.partial(
        pl.kernel,
        mesh=mesh,
        out_type=jax.ShapeDtypeStruct((t, d), jnp.float32),
        scratch_types=[
            pltpu.VMEM((2 * tpw,), jnp.int32),
            pltpu.VMEM((2 * _CC, d), jnp.float32),
            pltpu.VMEM((2 * _CC, d), jnp.float32),
            pltpu.VMEM((_CC, d), jnp.float32),
            pltpu.VMEM((_CC, d), jnp.float32),
            pltpu.SemaphoreType.DMA,
            pltpu.SemaphoreType.DMA,
            pltpu.SemaphoreType.DMA,
            pltpu.SemaphoreType.DMA,
        ],
    )
    def combine(ys_hbm, pos_hbm, out_hbm, idx_v, ga0, ga1, o0, o1,
                sg0, sg1, so0, so1):
        wid = lax.axis_index("s") * _NC + lax.axis_index("c")
        base = wid * tpw
        pltpu.sync_copy(pos_hbm.at[pl.ds(2 * base, 2 * tpw)], idx_v)
        gbufs = (ga0, ga1)
        obufs = (o0, o1)
        gsem = (sg0, sg1)
        osem = (so0, so1)
        gcp = [None, None]
        ocp = [None, None]
        gcp[0] = pltpu.async_copy(
            ys_hbm.at[idx_v.at[pl.ds(0, 2 * _CC)]], gbufs[0], gsem[0])
        for c in range(ncc):
            p = c & 1
            q = 1 - p
            if c + 1 < ncc:
                gcp[q] = pltpu.async_copy(
                    ys_hbm.at[idx_v.at[pl.ds((c + 1) * 2 * _CC, 2 * _CC)]],
                    gbufs[q], gsem[q])
            gcp[p].wait()
            if ocp[p] is not None:
                ocp[p].wait()
            gb = gbufs[p]
            ob = obufs[p]
            for r in range(_CC):
                @plsc.parallel_loop(0, nvec, unroll=8)
                def _add(i, gb=gb, ob=ob, r=r):
                    sl = pl.ds(i * 16, 16)
                    ob[r, sl] = gb[2 * r, sl] + gb[2 * r + 1, sl]
            ocp[p] = pltpu.async_copy(
                ob, out_hbm.at[pl.ds(base + c * _CC, _CC)], osem[p])
        for p in (0, 1):
            if ocp[p] is not None:
                ocp[p].wait()

    out = combine(ys, pos)
    return out.reshape(b, s, d)


# speedup vs baseline: 1.5553x; 1.5553x over previous
import functools

import jax
import jax.numpy as jnp
from jax import lax
from jax.experimental import pallas as pl
from jax.experimental.pallas import tpu as pltpu
from jax.experimental.pallas import tpu_sc as plsc

_NC, _NS = 2, 16
_NW = _NC * _NS

_BT = 256
_BTG = 512
_GC = 40
_CC = 16


def _gate_body(e_static, x_ref, wg_ref, e0_ref, e1_ref, w0_ref, w1_ref):
    logits = jnp.dot(x_ref[...], wg_ref[...], preferred_element_type=jnp.float32)
    col = lax.broadcasted_iota(jnp.int32, logits.shape, 1)
    neg = jnp.float32(-1e30)
    l0 = jnp.where(col < e_static, logits, neg)
    m1 = jnp.max(l0, axis=1, keepdims=True)
    i1 = jnp.min(jnp.where(l0 == m1, col, logits.shape[1]), axis=1, keepdims=True)
    lm = jnp.where(col == i1, neg, l0)
    m2 = jnp.max(lm, axis=1, keepdims=True)
    i2 = jnp.min(jnp.where(lm == m2, col, logits.shape[1]), axis=1, keepdims=True)
    t = jnp.exp(m2 - m1)
    w0 = 1.0 / (1.0 + t)
    e0_ref[0, 0, :] = i1[:, 0]
    e1_ref[0, 0, :] = i2[:, 0]
    w0_ref[0, 0, :] = w0[:, 0]
    w1_ref[0, 0, :] = 1.0 - w0[:, 0]


def _ffn_body(be_ref, xs_ref, w1_ref, b1_ref, w2_ref, b2_ref, ws_ref, ys_ref):
    xb = xs_ref[...].astype(jnp.bfloat16)
    h = jnp.dot(xb, w1_ref[0], preferred_element_type=jnp.float32) + b1_ref[0]
    g = jax.nn.gelu(h)
    y = jnp.dot(g.astype(jnp.bfloat16), w2_ref[0],
                preferred_element_type=jnp.float32) + b2_ref[0]
    ys_ref[...] = y * ws_ref[0, 0][:, None]


def _gating(xf, Wg):
    t, d = xf.shape
    e = Wg.shape[1]
    lanes = 128
    wg_pad = jnp.pad(Wg, ((0, 0), (0, lanes - e)))
    nbg = t // _BTG
    e0, e1, w0, w1 = pl.pallas_call(
        functools.partial(_gate_body, e),
        grid=(nbg,),
        in_specs=[
            pl.BlockSpec((_BTG, d), lambda i: (i, 0)),
            pl.BlockSpec((d, lanes), lambda i: (0, 0)),
        ],
        out_specs=[
            pl.BlockSpec((1, 1, _BTG), lambda i: (i, 0, 0)),
            pl.BlockSpec((1, 1, _BTG), lambda i: (i, 0, 0)),
            pl.BlockSpec((1, 1, _BTG), lambda i: (i, 0, 0)),
            pl.BlockSpec((1, 1, _BTG), lambda i: (i, 0, 0)),
        ],
        out_shape=[
            jax.ShapeDtypeStruct((nbg, 1, _BTG), jnp.int32),
            jax.ShapeDtypeStruct((nbg, 1, _BTG), jnp.int32),
            jax.ShapeDtypeStruct((nbg, 1, _BTG), jnp.float32),
            jax.ShapeDtypeStruct((nbg, 1, _BTG), jnp.float32),
        ],
    )(xf, wg_pad)
    return e0.reshape(t), e1.reshape(t), w0.reshape(t), w1.reshape(t)


def _route(e0, e1, w0, w1, e):
    t = e0.shape[0]
    p = t * 2
    nb = p // _BT + e
    npos = nb * _BT
    flat_e = jnp.stack([e0, e1], axis=1).reshape(p)
    flat_w = jnp.stack([w0, w1], axis=1).reshape(p)
    oh = (flat_e[:, None] == jnp.arange(e, dtype=jnp.int32)[None, :]).astype(jnp.int32)
    csum = jnp.cumsum(oh, axis=0)
    counts = csum[-1]
    rank = jnp.take_along_axis(csum, flat_e[:, None], axis=1)[:, 0] - 1
    nblk = (counts + _BT - 1) // _BT
    blk_start = jnp.concatenate(
        [jnp.zeros((1,), jnp.int32), jnp.cumsum(nblk)[:-1].astype(jnp.int32)])
    pos = blk_start[flat_e] * _BT + rank
    tok = jnp.arange(p, dtype=jnp.int32) // 2
    tok_pos = jnp.zeros((npos,), jnp.int32).at[pos].set(tok)
    ws_pos = jnp.zeros((npos,), jnp.float32).at[pos].set(flat_w)
    block_expert = jnp.minimum(
        (jnp.arange(nb, dtype=jnp.int32)[:, None] >= blk_start[None, :])
        .astype(jnp.int32).sum(axis=1) - 1,
        e - 1).astype(jnp.int32)
    return tok_pos, ws_pos, block_expert, pos


def _ffn(block_expert, xs, W1, b1, W2, b2, ws_pos):
    npos, d = xs.shape
    e, _, f = W1.shape
    nb = npos // _BT
    ws3 = ws_pos.reshape(nb, 1, _BT)
    grid_spec = pltpu.PrefetchScalarGridSpec(
        num_scalar_prefetch=1,
        grid=(nb,),
        in_specs=[
            pl.BlockSpec((_BT, d), lambda i, be: (i, 0)),
            pl.BlockSpec((1, d, f), lambda i, be: (be[i], 0, 0)),
            pl.BlockSpec((1, 1, f), lambda i, be: (be[i], 0, 0)),
            pl.BlockSpec((1, f, d), lambda i, be: (be[i], 0, 0)),
            pl.BlockSpec((1, 1, d), lambda i, be: (be[i], 0, 0)),
            pl.BlockSpec((1, 1, _BT), lambda i, be: (i, 0, 0)),
        ],
        out_specs=pl.BlockSpec((_BT, d), lambda i, be: (i, 0)),
    )
    return pl.pallas_call(
        _ffn_body,
        grid_spec=grid_spec,
        out_shape=jax.ShapeDtypeStruct((npos, d), jnp.float32),
    )(block_expert, xs, W1.astype(jnp.bfloat16), b1.reshape(e, 1, f),
      W2.astype(jnp.bfloat16), b2.reshape(e, 1, d), ws3)


def kernel(x, Wg, W1, b1, W2, b2):
    b, s, d = x.shape
    t = b * s
    e = Wg.shape[1]
    p = t * 2
    nb = p // _BT + e
    npos = nb * _BT

    xf = jnp.reshape(x, (t, d))

    e0, e1, w0, w1 = _gating(xf, Wg)

    tok_pos, ws_pos, block_expert, pos = _route(e0, e1, w0, w1, e)

    mesh = plsc.VectorSubcoreMesh(
        core_axis_name="c", subcore_axis_name="s",
        num_cores=_NC, num_subcores=_NS)
    per_w = npos // _NW
    nch = per_w // _GC

    @functools.partial(
        pl.kernel,
        mesh=mesh,
        out_type=jax.ShapeDtypeStruct((npos, d), jnp.float32),
        scratch_types=[
            pltpu.VMEM((per_w,), jnp.int32),
            pltpu.VMEM((_GC, d), jnp.float32),
            pltpu.VMEM((_GC, d), jnp.float32),
            pltpu.SemaphoreType.DMA,
            pltpu.SemaphoreType.DMA,
            pltpu.SemaphoreType.DMA,
            pltpu.SemaphoreType.DMA,
        ],
    )
    def dispatch(x_hbm, idx_hbm, xs_hbm, idx_v, buf0, buf1, g0, g1, w0s, w1s):
        wid = lax.axis_index("s") * _NC + lax.axis_index("c")
        base = wid * per_w
        pltpu.sync_copy(idx_hbm.at[pl.ds(base, per_w)], idx_v)
        bufs = (buf0, buf1)
        gsem = (g0, g1)
        wsem = (w0s, w1s)
        gcp = [None, None]
        wcp = [None, None]
        for c in range(nch):
            p = c & 1
            if wcp[p] is not None:
                wcp[p].wait()
            gcp[p] = pltpu.async_copy(
                x_hbm.at[idx_v.at[pl.ds(c * _GC, _GC)]], bufs[p], gsem[p])
            if c >= 1:
                q = 1 - p
                gcp[q].wait()
                wcp[q] = pltpu.async_copy(
                    bufs[q], xs_hbm.at[pl.ds(base + (c - 1) * _GC, _GC)], wsem[q])
        pl_ = (nch - 1) & 1
        gcp[pl_].wait()
        wcp[pl_] = pltpu.async_copy(
            bufs[pl_], xs_hbm.at[pl.ds(base + (nch - 1) * _GC, _GC)], wsem[pl_])
        for p in (0, 1):
            if wcp[p] is not None:
                wcp[p].wait()

    xs = dispatch(xf, tok_pos)

    ys = _ffn(block_expert, xs, W1, b1, W2, b2, ws_pos)

    tpw = t // _NW
    ncc = tpw // _CC
    nvec = d // 16

    @functools.partial(
        pl.kernel,
        mesh=mesh,
        out_type=jax.ShapeDtypeStruct((t, d), jnp.float32),
        scratch_types=[
            pltpu.VMEM((2 * tpw,), jnp.int32),
            pltpu.VMEM((2 * _CC, d), jnp.float32),
            pltpu.VMEM((2 * _CC, d), jnp.float32),
            pltpu.VMEM((_CC, d), jnp.float32),
            pltpu.VMEM((_CC, d), jnp.float32),
            pltpu.SemaphoreType.DMA,
            pltpu.SemaphoreType.DMA,
            pltpu.SemaphoreType.DMA,
            pltpu.SemaphoreType.DMA,
        ],
    )
    def combine(ys_hbm, pos_hbm, out_hbm, idx_v, ga0, ga1, o0, o1,
                sg0, sg1, so0, so1):
        wid = lax.axis_index("s") * _NC + lax.axis_index("c")
        base = wid * tpw
        pltpu.sync_copy(pos_hbm.at[pl.ds(2 * base, 2 * tpw)], idx_v)
        gbufs = (ga0, ga1)
        obufs = (o0, o1)
        gsem = (sg0, sg1)
        osem = (so0, so1)
        gcp = [None, None]
        ocp = [None, None]
        gcp[0] = pltpu.async_copy(
            ys_hbm.at[idx_v.at[pl.ds(0, 2 * _CC)]], gbufs[0], gsem[0])
        for c in range(ncc):
            p = c & 1
            q = 1 - p
            if c + 1 < ncc:
                gcp[q] = pltpu.async_copy(
                    ys_hbm.at[idx_v.at[pl.ds((c + 1) * 2 * _CC, 2 * _CC)]],
                    gbufs[q], gsem[q])
            gcp[p].wait()
            if ocp[p] is not None:
                ocp[p].wait()
            gb = gbufs[p]
            ob = obufs[p]
            for r in range(_CC):
                @plsc.parallel_loop(0, nvec, unroll=8)
                def _add(i, gb=gb, ob=ob, r=r):
                    sl = pl.ds(i * 16, 16)
                    ob[r, sl] = gb[2 * r, sl] + gb[2 * r + 1, sl]
            ocp[p] = pltpu.async_copy(
                ob, out_hbm.at[pl.ds(base + c * _CC, _CC)], osem[p])
        for p in (0, 1):
            if ocp[p] is not None:
                ocp[p].wait()

    out = combine(ys, pos)
    return out.reshape(b, s, d)


# speedup vs baseline: 1.7891x; 1.1503x over previous
import functools

import jax
import jax.numpy as jnp
from jax import lax
from jax.experimental import pallas as pl
from jax.experimental.pallas import tpu as pltpu
from jax.experimental.pallas import tpu_sc as plsc

_NC, _NS = 2, 16
_NW = _NC * _NS

_BT = 256
_BTG = 512
_GC = 40
_CC = 16
_NSEG = 4


def _gate_body(e_static, x_ref, wg_ref, e0_ref, e1_ref, w0_ref, w1_ref):
    logits = jnp.dot(x_ref[...], wg_ref[...], preferred_element_type=jnp.float32)
    col = lax.broadcasted_iota(jnp.int32, logits.shape, 1)
    neg = jnp.float32(-1e30)
    l0 = jnp.where(col < e_static, logits, neg)
    m1 = jnp.max(l0, axis=1, keepdims=True)
    i1 = jnp.min(jnp.where(l0 == m1, col, logits.shape[1]), axis=1, keepdims=True)
    lm = jnp.where(col == i1, neg, l0)
    m2 = jnp.max(lm, axis=1, keepdims=True)
    i2 = jnp.min(jnp.where(lm == m2, col, logits.shape[1]), axis=1, keepdims=True)
    t = jnp.exp(m2 - m1)
    w0 = 1.0 / (1.0 + t)
    e0_ref[0, 0, :] = i1[:, 0]
    e1_ref[0, 0, :] = i2[:, 0]
    w0_ref[0, 0, :] = w0[:, 0]
    w1_ref[0, 0, :] = 1.0 - w0[:, 0]


def _gating(xf, Wg):
    t, d = xf.shape
    e = Wg.shape[1]
    lanes = 128
    wg_pad = jnp.pad(Wg, ((0, 0), (0, lanes - e)))
    nbg = t // _BTG
    e0, e1, w0, w1 = pl.pallas_call(
        functools.partial(_gate_body, e),
        grid=(nbg,),
        in_specs=[
            pl.BlockSpec((_BTG, d), lambda i: (i, 0)),
            pl.BlockSpec((d, lanes), lambda i: (0, 0)),
        ],
        out_specs=[
            pl.BlockSpec((1, 1, _BTG), lambda i: (i, 0, 0)),
            pl.BlockSpec((1, 1, _BTG), lambda i: (i, 0, 0)),
            pl.BlockSpec((1, 1, _BTG), lambda i: (i, 0, 0)),
            pl.BlockSpec((1, 1, _BTG), lambda i: (i, 0, 0)),
        ],
        out_shape=[
            jax.ShapeDtypeStruct((nbg, 1, _BTG), jnp.int32),
            jax.ShapeDtypeStruct((nbg, 1, _BTG), jnp.int32),
            jax.ShapeDtypeStruct((nbg, 1, _BTG), jnp.float32),
            jax.ShapeDtypeStruct((nbg, 1, _BTG), jnp.float32),
        ],
    )(xf, wg_pad)
    return e0.reshape(t), e1.reshape(t), w0.reshape(t), w1.reshape(t)


def _route(e0, e1, w0, w1, e):
    t = e0.shape[0]
    p = t * 2
    nb = p // _BT + e
    npos = nb * _BT
    flat_e = jnp.stack([e0, e1], axis=1).reshape(p)
    flat_w = jnp.stack([w0, w1], axis=1).reshape(p)
    oh = (flat_e[:, None] == jnp.arange(e, dtype=jnp.int32)[None, :]).astype(jnp.int32)
    csum = jnp.cumsum(oh, axis=0)
    counts = csum[-1]
    rank = jnp.take_along_axis(csum, flat_e[:, None], axis=1)[:, 0] - 1
    nblk = (counts + _BT - 1) // _BT
    blk_start = jnp.concatenate(
        [jnp.zeros((1,), jnp.int32), jnp.cumsum(nblk)[:-1].astype(jnp.int32)])
    pos = blk_start[flat_e] * _BT + rank
    tok = jnp.arange(p, dtype=jnp.int32) // 2
    tok_pos = jnp.zeros((npos,), jnp.int32).at[pos].set(tok)
    ws_pos = jnp.zeros((npos,), jnp.float32).at[pos].set(flat_w)
    block_expert = jnp.minimum(
        (jnp.arange(nb, dtype=jnp.int32)[:, None] >= blk_start[None, :])
        .astype(jnp.int32).sum(axis=1) - 1,
        e - 1).astype(jnp.int32)
    return tok_pos, ws_pos, block_expert, pos


def _ffn_body(be_ref, xs_ref, w1_ref, b1_ref, w2_ref, b2_ref, ws_ref, ys_in_ref,
              ys_ref):
    xb = xs_ref[...]
    h = jnp.dot(xb, w1_ref[0], preferred_element_type=jnp.float32) + b1_ref[0]
    g = jax.nn.gelu(h)
    y = jnp.dot(g, w2_ref[0], preferred_element_type=jnp.float32) + b2_ref[0]
    ys_ref[...] = y * ws_ref[0, 0][:, None]


def _ffn_seg(be_seg, xs_seg, W1, b1r, W2, b2r, ws3_seg, ys_in, seg_off):
    nbseg, d = xs_seg.shape
    nbseg //= _BT
    e, _, f = W1.shape
    npos = ys_in.shape[0]
    grid_spec = pltpu.PrefetchScalarGridSpec(
        num_scalar_prefetch=1,
        grid=(nbseg,),
        in_specs=[
            pl.BlockSpec((_BT, d), lambda i, be: (i, 0)),
            pl.BlockSpec((1, d, f), lambda i, be: (be[i], 0, 0)),
            pl.BlockSpec((1, 1, f), lambda i, be: (be[i], 0, 0)),
            pl.BlockSpec((1, f, d), lambda i, be: (be[i], 0, 0)),
            pl.BlockSpec((1, 1, d), lambda i, be: (be[i], 0, 0)),
            pl.BlockSpec((1, 1, _BT), lambda i, be: (i, 0, 0)),
            pl.BlockSpec(memory_space=pl.ANY),
        ],
        out_specs=pl.BlockSpec((_BT, d), lambda i, be: (i + seg_off, 0)),
    )
    return pl.pallas_call(
        _ffn_body,
        grid_spec=grid_spec,
        out_shape=jax.ShapeDtypeStruct((npos, d), jnp.float32),
        input_output_aliases={7: 0},
    )(be_seg, xs_seg, W1, b1r, W2, b2r, ws3_seg, ys_in)


def kernel(x, Wg, W1, b1, W2, b2):
    b, s, d = x.shape
    t = b * s
    e = Wg.shape[1]
    f = W1.shape[2]
    p = t * 2
    nb = p // _BT + e
    npos = nb * _BT

    xf = jnp.reshape(x, (t, d))

    e0, e1, w0, w1 = _gating(xf, Wg)

    tok_pos, ws_pos, block_expert, pos = _route(e0, e1, w0, w1, e)

    mesh = plsc.VectorSubcoreMesh(
        core_axis_name="c", subcore_axis_name="s",
        num_cores=_NC, num_subcores=_NS)
    pseg = npos // _NSEG
    per_w = pseg // _NW
    nch = per_w // _GC

    @functools.partial(
        pl.kernel,
        mesh=mesh,
        out_type=jax.ShapeDtypeStruct((pseg, d), jnp.float32),
        scratch_types=[
            pltpu.VMEM((per_w,), jnp.int32),
            pltpu.VMEM((_GC, d), jnp.float32),
            pltpu.VMEM((_GC, d), jnp.float32),
            pltpu.SemaphoreType.DMA,
            pltpu.SemaphoreType.DMA,
            pltpu.SemaphoreType.DMA,
            pltpu.SemaphoreType.DMA,
        ],
    )
    def dispatch(x_hbm, idx_hbm, xs_hbm, idx_v, buf0, buf1, g0, g1, w0s, w1s):
        wid = lax.axis_index("s") * _NC + lax.axis_index("c")
        base = wid * per_w
        pltpu.sync_copy(idx_hbm.at[pl.ds(base, per_w)], idx_v)
        bufs = (buf0, buf1)
        gsem = (g0, g1)
        wsem = (w0s, w1s)
        gcp = [None, None]
        wcp = [None, None]
        for c in range(nch):
            pp = c & 1
            if wcp[pp] is not None:
                wcp[pp].wait()
            gcp[pp] = pltpu.async_copy(
                x_hbm.at[idx_v.at[pl.ds(c * _GC, _GC)]], bufs[pp], gsem[pp])
            if c >= 1:
                q = 1 - pp
                gcp[q].wait()
                wcp[q] = pltpu.async_copy(
                    bufs[q], xs_hbm.at[pl.ds(base + (c - 1) * _GC, _GC)], wsem[q])
        pl_ = (nch - 1) & 1
        gcp[pl_].wait()
        wcp[pl_] = pltpu.async_copy(
            bufs[pl_], xs_hbm.at[pl.ds(base + (nch - 1) * _GC, _GC)], wsem[pl_])
        for pp in (0, 1):
            if wcp[pp] is not None:
                wcp[pp].wait()

    b1r = b1.reshape(e, 1, f)
    b2r = b2.reshape(e, 1, d)
    ws3 = ws_pos.reshape(nb, 1, _BT)
    nbseg = nb // _NSEG
    xs_segs = []
    for sg in range(_NSEG):
        idx_seg = lax.slice(tok_pos, (sg * pseg,), ((sg + 1) * pseg,))
        xs_segs.append(dispatch(xf, idx_seg))

    ys = None
    for sg in range(_NSEG):
        be_seg = lax.slice(block_expert, (sg * nbseg,), ((sg + 1) * nbseg,))
        ws_seg = lax.slice(ws3, (sg * nbseg, 0, 0), ((sg + 1) * nbseg, 1, _BT))
        if ys is None:
            ys = jnp.zeros((npos, d), jnp.float32)
        ys = _ffn_seg(be_seg, xs_segs[sg], W1, b1r, W2, b2r, ws_seg, ys,
                      sg * nbseg)

    tpw = t // _NW
    ncc = tpw // _CC
    nvec = d // 16

    @functools.partial(
        pl.kernel,
        mesh=mesh,
        out_type=jax.ShapeDtypeStruct((t, d), jnp.float32),
        scratch_types=[
            pltpu.VMEM((2 * tpw,), jnp.int32),
            pltpu.VMEM((2 * _CC, d), jnp.float32),
            pltpu.VMEM((2 * _CC, d), jnp.float32),
            pltpu.VMEM((_CC, d), jnp.float32),
            pltpu.VMEM((_CC, d), jnp.float32),
            pltpu.SemaphoreType.DMA,
            pltpu.SemaphoreType.DMA,
            pltpu.SemaphoreType.DMA,
            pltpu.SemaphoreType.DMA,
        ],
    )
    def combine(ys_hbm, pos_hbm, out_hbm, idx_v, ga0, ga1, o0, o1,
                sg0, sg1, so0, so1):
        wid = lax.axis_index("s") * _NC + lax.axis_index("c")
        base = wid * tpw
        pltpu.sync_copy(pos_hbm.at[pl.ds(2 * base, 2 * tpw)], idx_v)
        gbufs = (ga0, ga1)
        obufs = (o0, o1)
        gsem = (sg0, sg1)
        osem = (so0, so1)
        gcp = [None, None]
        ocp = [None, None]
        gcp[0] = pltpu.async_copy(
            ys_hbm.at[idx_v.at[pl.ds(0, 2 * _CC)]], gbufs[0], gsem[0])
        for c in range(ncc):
            pp = c & 1
            q = 1 - pp
            if c + 1 < ncc:
                gcp[q] = pltpu.async_copy(
                    ys_hbm.at[idx_v.at[pl.ds((c + 1) * 2 * _CC, 2 * _CC)]],
                    gbufs[q], gsem[q])
            gcp[pp].wait()
            if ocp[pp] is not None:
                ocp[pp].wait()
            gb = gbufs[pp]
            ob = obufs[pp]
            for r in range(_CC):
                @plsc.parallel_loop(0, nvec, unroll=8)
                def _add(i, gb=gb, ob=ob, r=r):
                    sl = pl.ds(i * 16, 16)
                    ob[r, sl] = gb[2 * r, sl] + gb[2 * r + 1, sl]
            ocp[pp] = pltpu.async_copy(
                ob, out_hbm.at[pl.ds(base + c * _CC, _CC)], osem[pp])
        for pp in (0, 1):
            if ocp[pp] is not None:
                ocp[pp].wait()

    out = combine(ys, pos)
    return out.reshape(b, s, d)


# speedup vs baseline: 1.8774x; 1.0494x over previous
import functools

import jax
import jax.numpy as jnp
from jax import lax
from jax.experimental import pallas as pl
from jax.experimental.pallas import tpu as pltpu
from jax.experimental.pallas import tpu_sc as plsc

_NC, _NS = 2, 16
_NW = _NC * _NS

_BT = 256
_BTG = 512
_GC = 40
_CC = 16
_NSEG = 4


def _gate_body(e_static, x_ref, wg_ref, e0_ref, e1_ref, w0_ref, w1_ref):
    logits = jnp.dot(x_ref[...], wg_ref[...], preferred_element_type=jnp.float32)
    col = lax.broadcasted_iota(jnp.int32, logits.shape, 1)
    neg = jnp.float32(-1e30)
    l0 = jnp.where(col < e_static, logits, neg)
    m1 = jnp.max(l0, axis=1, keepdims=True)
    i1 = jnp.min(jnp.where(l0 == m1, col, logits.shape[1]), axis=1, keepdims=True)
    lm = jnp.where(col == i1, neg, l0)
    m2 = jnp.max(lm, axis=1, keepdims=True)
    i2 = jnp.min(jnp.where(lm == m2, col, logits.shape[1]), axis=1, keepdims=True)
    t = jnp.exp(m2 - m1)
    w0 = 1.0 / (1.0 + t)
    e0_ref[0, 0, :] = i1[:, 0]
    e1_ref[0, 0, :] = i2[:, 0]
    w0_ref[0, 0, :] = w0[:, 0]
    w1_ref[0, 0, :] = 1.0 - w0[:, 0]


def _gating(xf, Wg):
    t, d = xf.shape
    e = Wg.shape[1]
    lanes = 128
    wg_pad = jnp.pad(Wg, ((0, 0), (0, lanes - e)))
    nbg = t // _BTG
    e0, e1, w0, w1 = pl.pallas_call(
        functools.partial(_gate_body, e),
        grid=(nbg,),
        in_specs=[
            pl.BlockSpec((_BTG, d), lambda i: (i, 0)),
            pl.BlockSpec((d, lanes), lambda i: (0, 0)),
        ],
        out_specs=[
            pl.BlockSpec((1, 1, _BTG), lambda i: (i, 0, 0)),
            pl.BlockSpec((1, 1, _BTG), lambda i: (i, 0, 0)),
            pl.BlockSpec((1, 1, _BTG), lambda i: (i, 0, 0)),
            pl.BlockSpec((1, 1, _BTG), lambda i: (i, 0, 0)),
        ],
        out_shape=[
            jax.ShapeDtypeStruct((nbg, 1, _BTG), jnp.int32),
            jax.ShapeDtypeStruct((nbg, 1, _BTG), jnp.int32),
            jax.ShapeDtypeStruct((nbg, 1, _BTG), jnp.float32),
            jax.ShapeDtypeStruct((nbg, 1, _BTG), jnp.float32),
        ],
    )(xf, wg_pad)
    return e0.reshape(t), e1.reshape(t), w0.reshape(t), w1.reshape(t)


def _route(e0, e1, w0, w1, e):
    t = e0.shape[0]
    p = t * 2
    nb = p // _BT + e
    npos = nb * _BT
    flat_e = jnp.stack([e0, e1], axis=1).reshape(p)
    flat_w = jnp.stack([w0, w1], axis=1).reshape(p)
    oh = (flat_e[:, None] == jnp.arange(e, dtype=jnp.int32)[None, :]).astype(jnp.int32)
    csum = jnp.cumsum(oh, axis=0)
    counts = csum[-1]
    rank = jnp.take_along_axis(csum, flat_e[:, None], axis=1)[:, 0] - 1
    nblk = (counts + _BT - 1) // _BT
    blk_start = jnp.concatenate(
        [jnp.zeros((1,), jnp.int32), jnp.cumsum(nblk)[:-1].astype(jnp.int32)])
    pos = blk_start[flat_e] * _BT + rank
    tok = jnp.arange(p, dtype=jnp.int32) // 2
    tok_pos = jnp.zeros((npos,), jnp.int32).at[pos].set(tok)
    ws_pos = jnp.zeros((npos,), jnp.float32).at[pos].set(flat_w)
    block_expert = jnp.minimum(
        (jnp.arange(nb, dtype=jnp.int32)[:, None] >= blk_start[None, :])
        .astype(jnp.int32).sum(axis=1) - 1,
        e - 1).astype(jnp.int32)
    return tok_pos, ws_pos, block_expert, pos


def _ffn_body(be_ref, xs_ref, w1_ref, b1_ref, w2_ref, b2_ref, ws_ref,
              ys_in_ref=None, ys_ref=None, ys_in_skip=False):
    if ys_in_skip:
        ys_ref = ys_in_ref
    xb = xs_ref[...]
    h = jnp.dot(xb, w1_ref[0], preferred_element_type=jnp.float32) + b1_ref[0]
    g = jax.nn.gelu(h)
    y = jnp.dot(g, w2_ref[0], preferred_element_type=jnp.float32) + b2_ref[0]
    ys_ref[...] = y * ws_ref[0, 0][:, None]


def _ffn_seg(be_seg, xs_seg, W1, b1r, W2, b2r, ws3_seg, ys_in, seg_off, npos):
    nbseg, d = xs_seg.shape
    nbseg //= _BT
    e, _, f = W1.shape
    in_specs = [
        pl.BlockSpec((_BT, d), lambda i, be: (i, 0)),
        pl.BlockSpec((1, d, f), lambda i, be: (be[i], 0, 0)),
        pl.BlockSpec((1, 1, f), lambda i, be: (be[i], 0, 0)),
        pl.BlockSpec((1, f, d), lambda i, be: (be[i], 0, 0)),
        pl.BlockSpec((1, 1, d), lambda i, be: (be[i], 0, 0)),
        pl.BlockSpec((1, 1, _BT), lambda i, be: (i, 0, 0)),
    ]
    args = [be_seg, xs_seg, W1, b1r, W2, b2r, ws3_seg]
    aliases = {}
    body = _ffn_body
    if ys_in is not None:
        in_specs.append(pl.BlockSpec(memory_space=pl.ANY))
        args.append(ys_in)
        aliases = {7: 0}
    else:
        body = functools.partial(_ffn_body, ys_in_skip=True)
    grid_spec = pltpu.PrefetchScalarGridSpec(
        num_scalar_prefetch=1,
        grid=(nbseg,),
        in_specs=in_specs,
        out_specs=pl.BlockSpec((_BT, d), lambda i, be: (i + seg_off, 0)),
    )
    return pl.pallas_call(
        body,
        grid_spec=grid_spec,
        out_shape=jax.ShapeDtypeStruct((npos, d), jnp.float32),
        input_output_aliases=aliases,
    )(*args)


def kernel(x, Wg, W1, b1, W2, b2):
    b, s, d = x.shape
    t = b * s
    e = Wg.shape[1]
    f = W1.shape[2]
    p = t * 2
    nb = p // _BT + e
    npos = nb * _BT

    xf = jnp.reshape(x, (t, d))

    e0, e1, w0, w1 = _gating(xf, Wg)

    tok_pos, ws_pos, block_expert, pos = _route(e0, e1, w0, w1, e)

    mesh = plsc.VectorSubcoreMesh(
        core_axis_name="c", subcore_axis_name="s",
        num_cores=_NC, num_subcores=_NS)
    pseg = npos // _NSEG
    per_w = pseg // _NW
    nch = per_w // _GC

    @functools.partial(
        pl.kernel,
        mesh=mesh,
        out_type=jax.ShapeDtypeStruct((pseg, d), jnp.float32),
        scratch_types=[
            pltpu.VMEM((per_w,), jnp.int32),
            pltpu.VMEM((_GC, d), jnp.float32),
            pltpu.VMEM((_GC, d), jnp.float32),
            pltpu.SemaphoreType.DMA,
            pltpu.SemaphoreType.DMA,
            pltpu.SemaphoreType.DMA,
            pltpu.SemaphoreType.DMA,
        ],
    )
    def dispatch(x_hbm, idx_hbm, xs_hbm, idx_v, buf0, buf1, g0, g1, w0s, w1s):
        wid = lax.axis_index("s") * _NC + lax.axis_index("c")
        base = wid * per_w
        pltpu.sync_copy(idx_hbm.at[pl.ds(base, per_w)], idx_v)
        bufs = (buf0, buf1)
        gsem = (g0, g1)
        wsem = (w0s, w1s)
        gcp = [None, None]
        wcp = [None, None]
        for c in range(nch):
            pp = c & 1
            if wcp[pp] is not None:
                wcp[pp].wait()
            gcp[pp] = pltpu.async_copy(
                x_hbm.at[idx_v.at[pl.ds(c * _GC, _GC)]], bufs[pp], gsem[pp])
            if c >= 1:
                q = 1 - pp
                gcp[q].wait()
                wcp[q] = pltpu.async_copy(
                    bufs[q], xs_hbm.at[pl.ds(base + (c - 1) * _GC, _GC)], wsem[q])
        pl_ = (nch - 1) & 1
        gcp[pl_].wait()
        wcp[pl_] = pltpu.async_copy(
            bufs[pl_], xs_hbm.at[pl.ds(base + (nch - 1) * _GC, _GC)], wsem[pl_])
        for pp in (0, 1):
            if wcp[pp] is not None:
                wcp[pp].wait()

    b1r = b1.reshape(e, 1, f)
    b2r = b2.reshape(e, 1, d)
    ws3 = ws_pos.reshape(nb, 1, _BT)
    nbseg = nb // _NSEG
    xs_segs = []
    for sg in range(_NSEG):
        idx_seg = lax.slice(tok_pos, (sg * pseg,), ((sg + 1) * pseg,))
        xs_segs.append(dispatch(xf, idx_seg))

    ys = None
    for sg in range(_NSEG):
        be_seg = lax.slice(block_expert, (sg * nbseg,), ((sg + 1) * nbseg,))
        ws_seg = lax.slice(ws3, (sg * nbseg, 0, 0), ((sg + 1) * nbseg, 1, _BT))
        ys = _ffn_seg(be_seg, xs_segs[sg], W1, b1r, W2, b2r, ws_seg, ys,
                      sg * nbseg, npos)

    tpw = t // _NW
    ncc = tpw // _CC
    nvec = d // 16

    @functools.partial(
        pl.kernel,
        mesh=mesh,
        out_type=jax.ShapeDtypeStruct((t, d), jnp.float32),
        scratch_types=[
            pltpu.VMEM((2 * tpw,), jnp.int32),
            pltpu.VMEM((2 * _CC, d), jnp.float32),
            pltpu.VMEM((2 * _CC, d), jnp.float32),
            pltpu.VMEM((_CC, d), jnp.float32),
            pltpu.VMEM((_CC, d), jnp.float32),
            pltpu.SemaphoreType.DMA,
            pltpu.SemaphoreType.DMA,
            pltpu.SemaphoreType.DMA,
            pltpu.SemaphoreType.DMA,
        ],
    )
    def combine(ys_hbm, pos_hbm, out_hbm, idx_v, ga0, ga1, o0, o1,
                sg0, sg1, so0, so1):
        wid = lax.axis_index("s") * _NC + lax.axis_index("c")
        base = wid * tpw
        pltpu.sync_copy(pos_hbm.at[pl.ds(2 * base, 2 * tpw)], idx_v)
        gbufs = (ga0, ga1)
        obufs = (o0, o1)
        gsem = (sg0, sg1)
        osem = (so0, so1)
        gcp = [None, None]
        ocp = [None, None]
        gcp[0] = pltpu.async_copy(
            ys_hbm.at[idx_v.at[pl.ds(0, 2 * _CC)]], gbufs[0], gsem[0])
        for c in range(ncc):
            pp = c & 1
            q = 1 - pp
            if c + 1 < ncc:
                gcp[q] = pltpu.async_copy(
                    ys_hbm.at[idx_v.at[pl.ds((c + 1) * 2 * _CC, 2 * _CC)]],
                    gbufs[q], gsem[q])
            gcp[pp].wait()
            if ocp[pp] is not None:
                ocp[pp].wait()
            gb = gbufs[pp]
            ob = obufs[pp]
            for r in range(_CC):
                @plsc.parallel_loop(0, nvec, unroll=8)
                def _add(i, gb=gb, ob=ob, r=r):
                    sl = pl.ds(i * 16, 16)
                    ob[r, sl] = gb[2 * r, sl] + gb[2 * r + 1, sl]
            ocp[pp] = pltpu.async_copy(
                ob, out_hbm.at[pl.ds(base + c * _CC, _CC)], osem[pp])
        for pp in (0, 1):
            if ocp[pp] is not None:
                ocp[pp].wait()

    out = combine(ys, pos)
    return out.reshape(b, s, d)
